# Initial kernel scaffold; baseline (speedup 1.0000x reference)
import jax
import jax.numpy as jnp
from jax.experimental import pallas as pl


def kernel(x, edge_index, y, W1, b1, W2, b2, Wg1, as1, ad1, bg1, Wg2, as2, ad2, bg2):
    raise NotImplementedError("write your pallas kernel here")


# speedup vs baseline: 20.8519x; 20.8519x over previous
import jax
import jax.numpy as jnp
from jax import lax
from jax.experimental import pallas as pl
from jax.experimental.pallas import tpu as pltpu
from jax.experimental.pallas import tpu_sc as plsc

N = 10000
E = 320000
D = 128

NC = 2
NS = 16
L = 16
NW = NC * NS
EPT = E // NW
G = 80
NCHUNK = EPT // G
DW = 8
NZC = (N // G + NS - 1) // NS

NB = 2000
GRID = N // NB


def _tc1_body(x_ref, y_ref, w1_ref, b1_ref, w2_ref, b2_ref, wg_ref,
              avs_ref, avd_ref,
              g_ref, asrc_ref, adst_ref, shift_ref, acc_ref):
    i = pl.program_id(0)
    x = x_ref[...]
    h1 = jnp.maximum(x @ w1_ref[...] + b1_ref[...], 0.0)
    h2 = jnp.maximum(x @ w2_ref[...] + b2_ref[...], 0.0)
    h = jnp.where(y_ref[...] > 0.5, h1, h2)
    g = h @ wg_ref[...]
    g_ref[...] = g
    a_s = jnp.sum(g * avs_ref[...], axis=1, keepdims=True)
    a_d = jnp.sum(g * avd_ref[...], axis=1, keepdims=True)
    asrc_ref[...] = a_s
    adst_ref[...] = a_d

    ma = jnp.max(a_s)
    md = jnp.max(a_d)

    @pl.when(i == 0)
    def _():
        acc_ref[0] = ma
        acc_ref[1] = md

    @pl.when(i > 0)
    def _():
        acc_ref[0] = jnp.maximum(acc_ref[0], ma)
        acc_ref[1] = jnp.maximum(acc_ref[1], md)

    @pl.when(i == GRID - 1)
    def _():
        s = acc_ref[0] + acc_ref[1]
        shift_ref[...] = jnp.maximum(s, 0.2 * s).reshape(1, 1)


def _tc1(x, y_f, w1, b1, w2, b2, wg, avs, avd):
    row = lambda i: (i, 0)
    fixed = lambda i: (0, 0)
    return pl.pallas_call(
        _tc1_body,
        grid=(GRID,),
        in_specs=[
            pl.BlockSpec((NB, D), row),
            pl.BlockSpec((NB, 1), row),
            pl.BlockSpec((D, D), fixed),
            pl.BlockSpec((1, D), fixed),
            pl.BlockSpec((D, D), fixed),
            pl.BlockSpec((1, D), fixed),
            pl.BlockSpec((D, D), fixed),
            pl.BlockSpec((1, D), fixed),
            pl.BlockSpec((1, D), fixed),
        ],
        out_specs=[
            pl.BlockSpec((NB, D), row),
            pl.BlockSpec((NB, 1), row),
            pl.BlockSpec((NB, 1), row),
            pl.BlockSpec((1, 1), fixed),
        ],
        out_shape=[
            jax.ShapeDtypeStruct((N, D), jnp.float32),
            jax.ShapeDtypeStruct((N, 1), jnp.float32),
            jax.ShapeDtypeStruct((N, 1), jnp.float32),
            jax.ShapeDtypeStruct((1, 1), jnp.float32),
        ],
        scratch_shapes=[pltpu.SMEM((2,), jnp.float32)],
    )(x, y_f, w1, b1, w2, b2, wg, avs, avd)


def _tc2_body(accr_ref, accd_ref, bg_ref, wg_ref, avs_ref, avd_ref,
              g_ref, asrc_ref, adst_ref, shift_ref, macc_ref):
    i = pl.program_id(0)
    num = accr_ref[0] + accr_ref[1]
    dsum = accd_ref[0] + accd_ref[1]
    den = dsum[:, :1]
    o = num / (den + 1e-16) + bg_ref[...]
    h = jnp.where(o > 0.0, o, jnp.exp(jnp.minimum(o, 0.0)) - 1.0)
    g = h @ wg_ref[...]
    g_ref[...] = g
    a_s = jnp.sum(g * avs_ref[...], axis=1, keepdims=True)
    a_d = jnp.sum(g * avd_ref[...], axis=1, keepdims=True)
    asrc_ref[...] = a_s
    adst_ref[...] = a_d

    ma = jnp.max(a_s)
    md = jnp.max(a_d)

    @pl.when(i == 0)
    def _():
        macc_ref[0] = ma
        macc_ref[1] = md

    @pl.when(i > 0)
    def _():
        macc_ref[0] = jnp.maximum(macc_ref[0], ma)
        macc_ref[1] = jnp.maximum(macc_ref[1], md)

    @pl.when(i == GRID - 1)
    def _():
        s = macc_ref[0] + macc_ref[1]
        shift_ref[...] = jnp.maximum(s, 0.2 * s).reshape(1, 1)


def _tc2(accr, accd, bg, wg, avs, avd):
    row = lambda i: (i, 0)
    fixed = lambda i: (0, 0)
    return pl.pallas_call(
        _tc2_body,
        grid=(GRID,),
        in_specs=[
            pl.BlockSpec((2, NB, D), lambda i: (0, i, 0)),
            pl.BlockSpec((2, NB, DW), lambda i: (0, i, 0)),
            pl.BlockSpec((1, D), fixed),
            pl.BlockSpec((D, D), fixed),
            pl.BlockSpec((1, D), fixed),
            pl.BlockSpec((1, D), fixed),
        ],
        out_specs=[
            pl.BlockSpec((NB, D), row),
            pl.BlockSpec((NB, 1), row),
            pl.BlockSpec((NB, 1), row),
            pl.BlockSpec((1, 1), fixed),
        ],
        out_shape=[
            jax.ShapeDtypeStruct((N, D), jnp.float32),
            jax.ShapeDtypeStruct((N, 1), jnp.float32),
            jax.ShapeDtypeStruct((N, 1), jnp.float32),
            jax.ShapeDtypeStruct((1, 1), jnp.float32),
        ],
        scratch_shapes=[pltpu.SMEM((2,), jnp.float32)],
    )(accr, accd, bg, wg, avs, avd)


def _tc3_body(accr_ref, accd_ref, bg_ref, out_ref):
    num = accr_ref[0] + accr_ref[1]
    dsum = accd_ref[0] + accd_ref[1]
    den = dsum[:, :1]
    o = num / (den + 1e-16) + bg_ref[...]
    out_ref[...] = jnp.where(o > 0.0, o, jnp.exp(jnp.minimum(o, 0.0)) - 1.0)


def _tc3(accr, accd, bg):
    row = lambda i: (i, 0)
    fixed = lambda i: (0, 0)
    return pl.pallas_call(
        _tc3_body,
        grid=(GRID,),
        in_specs=[
            pl.BlockSpec((2, NB, D), lambda i: (0, i, 0)),
            pl.BlockSpec((2, NB, DW), lambda i: (0, i, 0)),
            pl.BlockSpec((1, D), fixed),
        ],
        out_specs=pl.BlockSpec((NB, D), row),
        out_shape=jax.ShapeDtypeStruct((N, D), jnp.float32),
    )(accr, accd, bg)


def _sc_body(g_h, asrc_h, adst_h, src_h, dst_h, shift_h,
             outr_h, outd_h,
             asrc_v, adst_v, sidx_v, didx_v, shift_v, ee_v,
             rows_v, den_v, accr_sh, accd_sh, sem):
    cid = lax.axis_index("c")
    sid = lax.axis_index("s")
    wid = sid * NC + cid

    pltpu.sync_copy(asrc_h, asrc_v)
    pltpu.sync_copy(adst_h, adst_v)
    pltpu.sync_copy(shift_h, shift_v)
    shift = shift_v[...]
    lane = lax.iota(jnp.int32, L)

    def _zrow(i, c):
        for k in range(D // L):
            rows_v[i, pl.ds(k * L, L)] = jnp.zeros((L,), jnp.float32)
        return c

    lax.fori_loop(0, G, _zrow, 0)
    zv = jnp.zeros((L,), jnp.float32)
    for i in range(G * DW // L):
        plsc.store_scatter(den_v, [2 * i + (lane >> 3), lane & 7], zv)

    for k in range(NZC):
        idx = sid + k * NS

        @pl.when(idx < N // G)
        def _():
            off = idx * G
            pltpu.sync_copy(rows_v, accr_sh.at[pl.ds(off, G)])
            pltpu.sync_copy(den_v, accd_sh.at[pl.ds(off, G)])

    plsc.subcore_barrier()

    def _chunk(j, carry):
        pltpu.sync_copy(src_h.at[wid, j], sidx_v)
        pltpu.sync_copy(dst_h.at[wid, j], didx_v)
        pltpu.async_copy(g_h.at[sidx_v], rows_v, sem).wait()
        for t in range(G // L):
            sidx = sidx_v[pl.ds(t * L, L)]
            didx = didx_v[pl.ds(t * L, L)]
            a1 = plsc.load_gather(asrc_v, [sidx])
            a2 = plsc.load_gather(adst_v, [didx])
            z = a1 + a2
            e = jnp.maximum(z, 0.2 * z)
            ee = jnp.exp(e - shift)
            ee_v[pl.ds(t * L, L)] = ee
            plsc.store_scatter(den_v, [lane + t * L, lane * 0], ee)

        def _row(r, c):
            sv = plsc.load_gather(ee_v, [jnp.full((L,), r, jnp.int32)])
            for k in range(D // L):
                rows_v[r, pl.ds(k * L, L)] = rows_v[r, pl.ds(k * L, L)] * sv
            return c

        lax.fori_loop(0, G, _row, 0)
        pltpu.sync_copy(rows_v, accr_sh.at[didx_v], add=True)
        pltpu.sync_copy(den_v, accd_sh.at[didx_v], add=True)
        return carry

    lax.fori_loop(0, NCHUNK, _chunk, 0)
    plsc.subcore_barrier()

    for k in range(NZC):
        idx = sid + k * NS

        @pl.when(idx < N // G)
        def _():
            off = idx * G
            pltpu.sync_copy(accr_sh.at[pl.ds(off, G)], rows_v)
            pltpu.sync_copy(rows_v, outr_h.at[cid, pl.ds(off, G)])
            pltpu.sync_copy(accd_sh.at[pl.ds(off, G)], den_v)
            pltpu.sync_copy(den_v, outd_h.at[cid, pl.ds(off, G)])


def _sc_layer(g, asrc, adst, src, dst, shift16):
    mesh = plsc.VectorSubcoreMesh(
        core_axis_name="c", subcore_axis_name="s", num_cores=NC,
        num_subcores=NS)
    f = pl.kernel(
        _sc_body,
        out_type=[
            jax.ShapeDtypeStruct((NC, N, D), jnp.float32),
            jax.ShapeDtypeStruct((NC, N, DW), jnp.float32),
        ],
        mesh=mesh,
        scratch_types=[
            pltpu.VMEM((N,), jnp.float32),
            pltpu.VMEM((N,), jnp.float32),
            pltpu.VMEM((G,), jnp.int32),
            pltpu.VMEM((G,), jnp.int32),
            pltpu.VMEM((L,), jnp.float32),
            pltpu.VMEM((G,), jnp.float32),
            pltpu.VMEM((G, D), jnp.float32),
            pltpu.VMEM((G, DW), jnp.float32),
            pltpu.VMEM_SHARED((N, D), jnp.float32),
            pltpu.VMEM_SHARED((N, DW), jnp.float32),
            pltpu.SemaphoreType.DMA,
        ],
        compiler_params=pltpu.CompilerParams(
            use_tc_tiling_on_sc=False, needs_layout_passes=False),
    )
    return f(g, asrc, adst, src, dst, shift16)


def kernel(x, edge_index, y, W1, b1, W2, b2,
           Wg1, as1, ad1, bg1, Wg2, as2, ad2, bg2):
    y_f = y.astype(jnp.float32).reshape(N, 1)
    src = edge_index[0].astype(jnp.int32).reshape(NW, NCHUNK, G)
    dst = edge_index[1].astype(jnp.int32).reshape(NW, NCHUNK, G)

    g1, asrc1, adst1, shift1 = _tc1(
        x, y_f, W1, b1.reshape(1, D), W2, b2.reshape(1, D), Wg1,
        as1.reshape(1, D), ad1.reshape(1, D))
    s16 = jnp.full((L,), shift1[0, 0], jnp.float32)
    accr1, accd1 = _sc_layer(
        g1, asrc1.reshape(N), adst1.reshape(N), src, dst, s16)

    g2, asrc2, adst2, shift2 = _tc2(
        accr1, accd1, bg1.reshape(1, D), Wg2,
        as2.reshape(1, D), ad2.reshape(1, D))
    s16b = jnp.full((L,), shift2[0, 0], jnp.float32)
    accr2, accd2 = _sc_layer(
        g2, asrc2.reshape(N), adst2.reshape(N), src, dst, s16b)

    return _tc3(accr2, accd2, bg2.reshape(1, D))


# speedup vs baseline: 43.3588x; 2.0794x over previous
import jax
import jax.numpy as jnp
from jax import lax
from jax.experimental import pallas as pl
from jax.experimental.pallas import tpu as pltpu
from jax.experimental.pallas import tpu_sc as plsc

N = 10000
E = 320000
D = 128

NC = 2
NS = 16
L = 16
NW = NC * NS
EPT = E // NW
G = 80
NCHUNK = EPT // G
AW = D + L
NZC = (N // G + NS - 1) // NS

NB = 2000
GRID = N // NB


def _tc_tail(i, g, avs_ref, avd_ref, gaug_ref, adst_ref, shift_ref, acc_ref):
    a_s = jnp.sum(g * avs_ref[...], axis=1, keepdims=True)
    a_d = jnp.sum(g * avd_ref[...], axis=1, keepdims=True)
    col = lax.broadcasted_iota(jnp.int32, (NB, L), 1)
    gaug_ref[:, :D] = g
    gaug_ref[:, D:] = jnp.where(col == 0, 1.0, jnp.where(col == 1, a_s, 0.0))
    adst_ref[...] = jnp.broadcast_to(a_d, (NB, L))

    ma = jnp.max(a_s)
    md = jnp.max(a_d)

    @pl.when(i == 0)
    def _():
        acc_ref[0] = ma
        acc_ref[1] = md

    @pl.when(i > 0)
    def _():
        acc_ref[0] = jnp.maximum(acc_ref[0], ma)
        acc_ref[1] = jnp.maximum(acc_ref[1], md)

    @pl.when(i == GRID - 1)
    def _():
        s = acc_ref[0] + acc_ref[1]
        shift_ref[...] = jnp.maximum(s, 0.2 * s).reshape(1, 1)


_TC_OUT_SPECS = [
    pl.BlockSpec((NB, AW), lambda i: (i, 0)),
    pl.BlockSpec((NB, L), lambda i: (i, 0)),
    pl.BlockSpec((1, 1), lambda i: (0, 0)),
]
_TC_OUT_SHAPE = [
    jax.ShapeDtypeStruct((N, AW), jnp.float32),
    jax.ShapeDtypeStruct((N, L), jnp.float32),
    jax.ShapeDtypeStruct((1, 1), jnp.float32),
]


def _tc1_body(x_ref, y_ref, w1_ref, b1_ref, w2_ref, b2_ref, wg_ref,
              avs_ref, avd_ref,
              gaug_ref, adst_ref, shift_ref, acc_ref):
    i = pl.program_id(0)
    x = x_ref[...]
    h1 = jnp.maximum(x @ w1_ref[...] + b1_ref[...], 0.0)
    h2 = jnp.maximum(x @ w2_ref[...] + b2_ref[...], 0.0)
    h = jnp.where(y_ref[...] > 0.5, h1, h2)
    g = h @ wg_ref[...]
    _tc_tail(i, g, avs_ref, avd_ref, gaug_ref, adst_ref, shift_ref, acc_ref)


def _tc1(x, y_f, w1, b1, w2, b2, wg, avs, avd):
    row = lambda i: (i, 0)
    fixed = lambda i: (0, 0)
    return pl.pallas_call(
        _tc1_body,
        grid=(GRID,),
        in_specs=[
            pl.BlockSpec((NB, D), row),
            pl.BlockSpec((NB, 1), row),
            pl.BlockSpec((D, D), fixed),
            pl.BlockSpec((1, D), fixed),
            pl.BlockSpec((D, D), fixed),
            pl.BlockSpec((1, D), fixed),
            pl.BlockSpec((D, D), fixed),
            pl.BlockSpec((1, D), fixed),
            pl.BlockSpec((1, D), fixed),
        ],
        out_specs=_TC_OUT_SPECS,
        out_shape=_TC_OUT_SHAPE,
        scratch_shapes=[pltpu.SMEM((2,), jnp.float32)],
    )(x, y_f, w1, b1, w2, b2, wg, avs, avd)


def _tc2_body(accr_ref, bg_ref, wg_ref, avs_ref, avd_ref,
              gaug_ref, adst_ref, shift_ref, acc_ref):
    i = pl.program_id(0)
    p = accr_ref[0] + accr_ref[1]
    num = p[:, :D]
    den = p[:, D:D + 1]
    o = num / (den + 1e-16) + bg_ref[...]
    h = jnp.where(o > 0.0, o, jnp.exp(jnp.minimum(o, 0.0)) - 1.0)
    g = h @ wg_ref[...]
    _tc_tail(i, g, avs_ref, avd_ref, gaug_ref, adst_ref, shift_ref, acc_ref)


def _tc2(accr, bg, wg, avs, avd):
    fixed = lambda i: (0, 0)
    return pl.pallas_call(
        _tc2_body,
        grid=(GRID,),
        in_specs=[
            pl.BlockSpec((2, NB, AW), lambda i: (0, i, 0)),
            pl.BlockSpec((1, D), fixed),
            pl.BlockSpec((D, D), fixed),
            pl.BlockSpec((1, D), fixed),
            pl.BlockSpec((1, D), fixed),
        ],
        out_specs=_TC_OUT_SPECS,
        out_shape=_TC_OUT_SHAPE,
        scratch_shapes=[pltpu.SMEM((2,), jnp.float32)],
    )(accr, bg, wg, avs, avd)


def _tc3_body(accr_ref, bg_ref, out_ref):
    p = accr_ref[0] + accr_ref[1]
    num = p[:, :D]
    den = p[:, D:D + 1]
    o = num / (den + 1e-16) + bg_ref[...]
    out_ref[...] = jnp.where(o > 0.0, o, jnp.exp(jnp.minimum(o, 0.0)) - 1.0)


def _tc3(accr, bg):
    row = lambda i: (i, 0)
    fixed = lambda i: (0, 0)
    return pl.pallas_call(
        _tc3_body,
        grid=(GRID,),
        in_specs=[
            pl.BlockSpec((2, NB, AW), lambda i: (0, i, 0)),
            pl.BlockSpec((1, D), fixed),
        ],
        out_specs=pl.BlockSpec((NB, D), row),
        out_shape=jax.ShapeDtypeStruct((N, D), jnp.float32),
    )(accr, bg)


def _sc_body(gaug_h, adst_h, sd_h, shift_h,
             outr_h,
             rows0, rows1, rows2, avd0, avd1, avd2,
             idx0, idx1, idx2, idx3, idx4, idx5,
             ee_v, shift_v, accr_sh,
             sg0, sg1, sg2, sa0, sa1, sa2,
             si0, si1, si2, si3, si4, si5,
             ss0, ss1, ss2):
    rows = (rows0, rows1, rows2)
    avd = (avd0, avd1, avd2)
    idx = (idx0, idx1, idx2, idx3, idx4, idx5)
    sg = (sg0, sg1, sg2)
    sa = (sa0, sa1, sa2)
    si = (si0, si1, si2, si3, si4, si5)
    ss = (ss0, ss1, ss2)

    cid = lax.axis_index("c")
    sid = lax.axis_index("s")
    wid = sid * NC + cid

    pltpu.sync_copy(shift_h, shift_v)
    shift = shift_v[...]
    lane = lax.iota(jnp.int32, L)

    def _zrow(i, c):
        for k in range(AW // L):
            rows0[i, pl.ds(k * L, L)] = jnp.zeros((L,), jnp.float32)
        return c

    lax.fori_loop(0, G, _zrow, 0)
    for k in range(NZC):
        zi = sid + k * NS

        @pl.when(zi < N // G)
        def _():
            pltpu.sync_copy(rows0, accr_sh.at[pl.ds(zi * G, G)])

    plsc.subcore_barrier()

    def _issue_gather(k6, r3):
        pltpu.async_copy(gaug_h.at[idx[k6].at[0]], rows[r3], sg[r3])
        pltpu.async_copy(adst_h.at[idx[k6].at[1]], avd[r3], sa[r3])

    def _wait_gather(k6, r3):
        pltpu.make_async_copy(gaug_h.at[idx[k6].at[0]], rows[r3],
                              sg[r3]).wait()
        pltpu.make_async_copy(adst_h.at[idx[k6].at[1]], avd[r3],
                              sa[r3]).wait()

    def _wait_scatter(k6, r3):
        pltpu.make_async_copy(rows[r3], accr_sh.at[idx[k6].at[1]],
                              ss[r3]).wait()

    def _compute(r3):
        rr = rows[r3]
        av = avd[r3]
        for t in range(G // L):
            rowi = lane + t * L
            a1 = plsc.load_gather(rr, [rowi, lane * 0 + (D + 1)])
            a2 = plsc.load_gather(av, [rowi, lane * 0])
            z = a1 + a2
            e = jnp.maximum(z, 0.2 * z)
            ee_v[pl.ds(t * L, L)] = jnp.exp(e - shift)

        def _row(r, c):
            sv = plsc.load_gather(ee_v, [jnp.full((L,), r, jnp.int32)])
            for k in range(AW // L):
                rr[r, pl.ds(k * L, L)] = rr[r, pl.ds(k * L, L)] * sv
            return c

        lax.fori_loop(0, G, _row, 0)

    def _body(j, k6, r3):
        _wait_gather(k6, r3)
        _compute(r3)
        pltpu.async_copy(rows[r3], accr_sh.at[idx[k6].at[1]], ss[r3],
                         add=True)
        pltpu.async_copy(sd_h.at[wid, j + 4], idx[(k6 + 4) % 6],
                         si[(k6 + 4) % 6])
        @pl.when(j >= 1)
        def _():
            _wait_scatter((k6 + 5) % 6, (r3 + 2) % 3)
        pltpu.make_async_copy(sd_h.at[wid, 0], idx[(k6 + 2) % 6],
                              si[(k6 + 2) % 6]).wait()
        _issue_gather((k6 + 2) % 6, (r3 + 2) % 3)

    for p in range(2):
        pltpu.sync_copy(sd_h.at[wid, p], idx[p])
    for p in range(2, 4):
        pltpu.async_copy(sd_h.at[wid, p], idx[p], si[p])
    for p in range(2):
        _issue_gather(p, p)

    def _six(jo2, carry):
        for u in range(6):
            _body(jo2 * 6 + u, u, u % 3)
        return carry

    lax.fori_loop(0, 20, _six, 0)

    for j in range(120, NCHUNK):
        k6 = j % 6
        r3 = j % 3
        _wait_gather(k6, r3)
        _compute(r3)
        pltpu.async_copy(rows[r3], accr_sh.at[idx[k6].at[1]], ss[r3],
                         add=True)
        if j == 120:
            pltpu.async_copy(sd_h.at[wid, j + 4], idx[(k6 + 4) % 6],
                             si[(k6 + 4) % 6])
        _wait_scatter((k6 + 5) % 6, (r3 + 2) % 3)
        if j + 2 < NCHUNK:
            pltpu.make_async_copy(sd_h.at[wid, 0], idx[(k6 + 2) % 6],
                                  si[(k6 + 2) % 6]).wait()
            _issue_gather((k6 + 2) % 6, (r3 + 2) % 3)

    _wait_scatter((NCHUNK - 1) % 6, (NCHUNK - 1) % 3)
    plsc.subcore_barrier()

    for k in range(NZC):
        zi = sid + k * NS

        @pl.when(zi < N // G)
        def _():
            pltpu.sync_copy(accr_sh.at[pl.ds(zi * G, G)], rows0)
            pltpu.sync_copy(rows0, outr_h.at[cid, pl.ds(zi * G, G)])


def _sc_layer(gaug, adst16, sd, shift16):
    mesh = plsc.VectorSubcoreMesh(
        core_axis_name="c", subcore_axis_name="s", num_cores=NC,
        num_subcores=NS)
    f = pl.kernel(
        _sc_body,
        out_type=jax.ShapeDtypeStruct((NC, N, AW), jnp.float32),
        mesh=mesh,
        scratch_types=(
            [pltpu.VMEM((G, AW), jnp.float32)] * 3
            + [pltpu.VMEM((G, L), jnp.float32)] * 3
            + [pltpu.VMEM((2, G), jnp.int32)] * 6
            + [pltpu.VMEM((G,), jnp.float32),
               pltpu.VMEM((L,), jnp.float32),
               pltpu.VMEM_SHARED((N, AW), jnp.float32)]
            + [pltpu.SemaphoreType.DMA] * 15
        ),
        compiler_params=pltpu.CompilerParams(
            use_tc_tiling_on_sc=False, needs_layout_passes=False),
    )
    return f(gaug, adst16, sd, shift16)


def kernel(x, edge_index, y, W1, b1, W2, b2,
           Wg1, as1, ad1, bg1, Wg2, as2, ad2, bg2):
    y_f = y.astype(jnp.float32).reshape(N, 1)
    sd = (edge_index.astype(jnp.int32)
          .reshape(2, NW, NCHUNK, G).transpose(1, 2, 0, 3))

    gaug1, adst1, shift1 = _tc1(
        x, y_f, W1, b1.reshape(1, D), W2, b2.reshape(1, D), Wg1,
        as1.reshape(1, D), ad1.reshape(1, D))
    s16 = jnp.full((L,), shift1[0, 0], jnp.float32)
    accr1 = _sc_layer(gaug1, adst1, sd, s16)

    gaug2, adst2, shift2 = _tc2(
        accr1, bg1.reshape(1, D), Wg2, as2.reshape(1, D), ad2.reshape(1, D))
    s16b = jnp.full((L,), shift2[0, 0], jnp.float32)
    accr2 = _sc_layer(gaug2, adst2, sd, s16b)

    return _tc3(accr2, bg2.reshape(1, D))


# speedup vs baseline: 47.7787x; 1.1019x over previous
import jax
import jax.numpy as jnp
from jax import lax
from jax.experimental import pallas as pl
from jax.experimental.pallas import tpu as pltpu
from jax.experimental.pallas import tpu_sc as plsc

N = 10000
E = 320000
D = 128

NC = 2
NS = 16
L = 16
NW = NC * NS
EPT = E // NW
G = 80
NCHUNK = EPT // G
AW = D + L
NZC = (N // G + NS - 1) // NS

NB = 2000
GRID = N // NB


def _tc_tail(i, g, avs_ref, avd_ref, gaug_ref, adst_ref, shift_ref, acc_ref):
    a_s = jnp.sum(g * avs_ref[...], axis=1, keepdims=True)
    a_d = jnp.sum(g * avd_ref[...], axis=1, keepdims=True)
    col = lax.broadcasted_iota(jnp.int32, (NB, L), 1)
    gaug_ref[:, :D] = g
    gaug_ref[:, D:] = jnp.where(col == 0, 1.0, jnp.where(col == 1, a_s, 0.0))
    adst_ref[...] = jnp.broadcast_to(a_d, (NB, L))

    ma = jnp.max(a_s)
    md = jnp.max(a_d)

    @pl.when(i == 0)
    def _():
        acc_ref[0] = ma
        acc_ref[1] = md

    @pl.when(i > 0)
    def _():
        acc_ref[0] = jnp.maximum(acc_ref[0], ma)
        acc_ref[1] = jnp.maximum(acc_ref[1], md)

    @pl.when(i == GRID - 1)
    def _():
        s = acc_ref[0] + acc_ref[1]
        shift_ref[...] = jnp.maximum(s, 0.2 * s).reshape(1, 1)


_TC_OUT_SPECS = [
    pl.BlockSpec((NB, AW), lambda i: (i, 0)),
    pl.BlockSpec((NB, L), lambda i: (i, 0)),
    pl.BlockSpec((1, 1), lambda i: (0, 0)),
]
_TC_OUT_SHAPE = [
    jax.ShapeDtypeStruct((N, AW), jnp.float32),
    jax.ShapeDtypeStruct((N, L), jnp.float32),
    jax.ShapeDtypeStruct((1, 1), jnp.float32),
]


def _tc1_body(x_ref, y_ref, w1_ref, b1_ref, w2_ref, b2_ref, wg_ref,
              avs_ref, avd_ref,
              gaug_ref, adst_ref, shift_ref, acc_ref):
    i = pl.program_id(0)
    x = x_ref[...]
    h1 = jnp.maximum(x @ w1_ref[...] + b1_ref[...], 0.0)
    h2 = jnp.maximum(x @ w2_ref[...] + b2_ref[...], 0.0)
    h = jnp.where(y_ref[...] > 0.5, h1, h2)
    g = h @ wg_ref[...]
    _tc_tail(i, g, avs_ref, avd_ref, gaug_ref, adst_ref, shift_ref, acc_ref)


def _tc1(x, y_f, w1, b1, w2, b2, wg, avs, avd):
    row = lambda i: (i, 0)
    fixed = lambda i: (0, 0)
    return pl.pallas_call(
        _tc1_body,
        grid=(GRID,),
        in_specs=[
            pl.BlockSpec((NB, D), row),
            pl.BlockSpec((NB, 1), row),
            pl.BlockSpec((D, D), fixed),
            pl.BlockSpec((1, D), fixed),
            pl.BlockSpec((D, D), fixed),
            pl.BlockSpec((1, D), fixed),
            pl.BlockSpec((D, D), fixed),
            pl.BlockSpec((1, D), fixed),
            pl.BlockSpec((1, D), fixed),
        ],
        out_specs=_TC_OUT_SPECS,
        out_shape=_TC_OUT_SHAPE,
        scratch_shapes=[pltpu.SMEM((2,), jnp.float32)],
    )(x, y_f, w1, b1, w2, b2, wg, avs, avd)


def _tc2_body(accr_ref, bg_ref, wg_ref, avs_ref, avd_ref,
              gaug_ref, adst_ref, shift_ref, acc_ref):
    i = pl.program_id(0)
    p = accr_ref[0] + accr_ref[1]
    num = p[:, :D]
    den = p[:, D:D + 1]
    o = num / (den + 1e-16) + bg_ref[...]
    h = jnp.where(o > 0.0, o, jnp.exp(jnp.minimum(o, 0.0)) - 1.0)
    g = h @ wg_ref[...]
    _tc_tail(i, g, avs_ref, avd_ref, gaug_ref, adst_ref, shift_ref, acc_ref)


def _tc2(accr, bg, wg, avs, avd):
    fixed = lambda i: (0, 0)
    return pl.pallas_call(
        _tc2_body,
        grid=(GRID,),
        in_specs=[
            pl.BlockSpec((2, NB, AW), lambda i: (0, i, 0)),
            pl.BlockSpec((1, D), fixed),
            pl.BlockSpec((D, D), fixed),
            pl.BlockSpec((1, D), fixed),
            pl.BlockSpec((1, D), fixed),
        ],
        out_specs=_TC_OUT_SPECS,
        out_shape=_TC_OUT_SHAPE,
        scratch_shapes=[pltpu.SMEM((2,), jnp.float32)],
    )(accr, bg, wg, avs, avd)


def _tc3_body(accr_ref, bg_ref, out_ref):
    p = accr_ref[0] + accr_ref[1]
    num = p[:, :D]
    den = p[:, D:D + 1]
    o = num / (den + 1e-16) + bg_ref[...]
    out_ref[...] = jnp.where(o > 0.0, o, jnp.exp(jnp.minimum(o, 0.0)) - 1.0)


def _tc3(accr, bg):
    row = lambda i: (i, 0)
    fixed = lambda i: (0, 0)
    return pl.pallas_call(
        _tc3_body,
        grid=(GRID,),
        in_specs=[
            pl.BlockSpec((2, NB, AW), lambda i: (0, i, 0)),
            pl.BlockSpec((1, D), fixed),
        ],
        out_specs=pl.BlockSpec((NB, D), row),
        out_shape=jax.ShapeDtypeStruct((N, D), jnp.float32),
    )(accr, bg)


def _sc_body(gaug_h, adst_h, sd_h, shift_h,
             outr_h,
             rows0, rows1, rows2, avd0, avd1, avd2,
             sx0, sx1, sx2, sx3, sx4, sx5,
             dx0, dx1, dx2, dx3, dx4, dx5,
             ee_v, shift_v, accr_sh,
             sg0, sg1, sg2, sa0, sa1, sa2,
             si0, si1, si2, si3, si4, si5,
             ss0, ss1, ss2):
    rows = (rows0, rows1, rows2)
    avd = (avd0, avd1, avd2)
    sx = (sx0, sx1, sx2, sx3, sx4, sx5)
    dx = (dx0, dx1, dx2, dx3, dx4, dx5)
    sg = (sg0, sg1, sg2)
    sa = (sa0, sa1, sa2)
    si = (si0, si1, si2, si3, si4, si5)
    ss = (ss0, ss1, ss2)

    cid = lax.axis_index("c")
    sid = lax.axis_index("s")
    wid = sid * NC + cid

    pltpu.sync_copy(shift_h, shift_v)
    shift = shift_v[...]
    lane = lax.iota(jnp.int32, L)

    def _zrow(i, c):
        for k in range(AW // L):
            rows0[i, pl.ds(k * L, L)] = jnp.zeros((L,), jnp.float32)
        return c

    lax.fori_loop(0, G, _zrow, 0)
    for k in range(NZC):
        zi = sid + k * NS

        @pl.when(zi < N // G)
        def _():
            pltpu.sync_copy(rows0, accr_sh.at[pl.ds(zi * G, G)])

    plsc.subcore_barrier()

    def _issue_idx(j, k6):
        pltpu.async_copy(sd_h.at[0, wid, j], sx[k6], si[k6])
        pltpu.async_copy(sd_h.at[1, wid, j], dx[k6], si[k6])

    def _wait_idx(k6):
        pltpu.make_async_copy(sd_h.at[0, wid, 0], sx[k6], si[k6]).wait()
        pltpu.make_async_copy(sd_h.at[1, wid, 0], dx[k6], si[k6]).wait()

    def _issue_gather(k6, r3):
        pltpu.async_copy(gaug_h.at[sx[k6]], rows[r3], sg[r3])
        pltpu.async_copy(adst_h.at[dx[k6]], avd[r3], sa[r3])

    def _wait_gather(k6, r3):
        pltpu.make_async_copy(gaug_h.at[sx[k6]], rows[r3],
                              sg[r3]).wait()
        pltpu.make_async_copy(adst_h.at[dx[k6]], avd[r3],
                              sa[r3]).wait()

    def _wait_scatter(k6, r3):
        pltpu.make_async_copy(rows[r3], accr_sh.at[dx[k6]],
                              ss[r3]).wait()

    def _compute(r3):
        rr = rows[r3]
        av = avd[r3]
        for t in range(G // L):
            rowi = lane + t * L
            a1 = plsc.load_gather(rr, [rowi, lane * 0 + (D + 1)])
            a2 = plsc.load_gather(av, [rowi, lane * 0])
            z = a1 + a2
            e = jnp.maximum(z, 0.2 * z)
            ee_v[pl.ds(t * L, L)] = jnp.exp(e - shift)

        @plsc.parallel_loop(0, G, step=1, unroll=4)
        def _row(r):
            sv = plsc.load_gather(ee_v, [jnp.full((L,), r, jnp.int32)])
            for k in range(AW // L):
                rr[r, pl.ds(k * L, L)] = rr[r, pl.ds(k * L, L)] * sv

    def _body(j, k6, r3):
        _wait_gather(k6, r3)
        _compute(r3)
        pltpu.async_copy(rows[r3], accr_sh.at[dx[k6]], ss[r3],
                         add=True)
        _issue_idx(j + 4, (k6 + 4) % 6)
        @pl.when(j >= 1)
        def _():
            _wait_scatter((k6 + 5) % 6, (r3 + 2) % 3)
        _wait_idx((k6 + 2) % 6)
        _issue_gather((k6 + 2) % 6, (r3 + 2) % 3)

    for p in range(2):
        pltpu.sync_copy(sd_h.at[0, wid, p], sx[p])
        pltpu.sync_copy(sd_h.at[1, wid, p], dx[p])
    for p in range(2, 4):
        _issue_idx(p, p)
    for p in range(2):
        _issue_gather(p, p)

    def _six(jo2, carry):
        for u in range(6):
            _body(jo2 * 6 + u, u, u % 3)
        return carry

    lax.fori_loop(0, 20, _six, 0)

    for j in range(120, NCHUNK):
        k6 = j % 6
        r3 = j % 3
        _wait_gather(k6, r3)
        _compute(r3)
        pltpu.async_copy(rows[r3], accr_sh.at[dx[k6]], ss[r3],
                         add=True)
        if j == 120:
            _issue_idx(j + 4, (k6 + 4) % 6)
        _wait_scatter((k6 + 5) % 6, (r3 + 2) % 3)
        if j + 2 < NCHUNK:
            _wait_idx((k6 + 2) % 6)
            _issue_gather((k6 + 2) % 6, (r3 + 2) % 3)

    _wait_scatter((NCHUNK - 1) % 6, (NCHUNK - 1) % 3)
    plsc.subcore_barrier()

    for k in range(NZC):
        zi = sid + k * NS

        @pl.when(zi < N // G)
        def _():
            pltpu.sync_copy(accr_sh.at[pl.ds(zi * G, G)], rows0)
            pltpu.sync_copy(rows0, outr_h.at[cid, pl.ds(zi * G, G)])


def _sc_layer(gaug, adst16, sd, shift16):
    mesh = plsc.VectorSubcoreMesh(
        core_axis_name="c", subcore_axis_name="s", num_cores=NC,
        num_subcores=NS)
    f = pl.kernel(
        _sc_body,
        out_type=jax.ShapeDtypeStruct((NC, N, AW), jnp.float32),
        mesh=mesh,
        scratch_types=(
            [pltpu.VMEM((G, AW), jnp.float32)] * 3
            + [pltpu.VMEM((G, L), jnp.float32)] * 3
            + [pltpu.VMEM((G,), jnp.int32)] * 12
            + [pltpu.VMEM((G,), jnp.float32),
               pltpu.VMEM((L,), jnp.float32),
               pltpu.VMEM_SHARED((N, AW), jnp.float32)]
            + [pltpu.SemaphoreType.DMA] * 15
        ),
        compiler_params=pltpu.CompilerParams(
            use_tc_tiling_on_sc=False, needs_layout_passes=False),
    )
    return f(gaug, adst16, sd, shift16)


def kernel(x, edge_index, y, W1, b1, W2, b2,
           Wg1, as1, ad1, bg1, Wg2, as2, ad2, bg2):
    y_f = y.astype(jnp.float32).reshape(N, 1)
    sd = edge_index.astype(jnp.int32).reshape(2, NW, NCHUNK, G)

    gaug1, adst1, shift1 = _tc1(
        x, y_f, W1, b1.reshape(1, D), W2, b2.reshape(1, D), Wg1,
        as1.reshape(1, D), ad1.reshape(1, D))
    s16 = jnp.full((L,), shift1[0, 0], jnp.float32)
    accr1 = _sc_layer(gaug1, adst1, sd, s16)

    gaug2, adst2, shift2 = _tc2(
        accr1, bg1.reshape(1, D), Wg2, as2.reshape(1, D), ad2.reshape(1, D))
    s16b = jnp.full((L,), shift2[0, 0], jnp.float32)
    accr2 = _sc_layer(gaug2, adst2, sd, s16b)

    return _tc3(accr2, bg2.reshape(1, D))


# speedup vs baseline: 53.4493x; 1.1187x over previous
import jax
import jax.numpy as jnp
from jax import lax
from jax.experimental import pallas as pl
from jax.experimental.pallas import tpu as pltpu
from jax.experimental.pallas import tpu_sc as plsc

N = 10000
E = 320000
D = 128

NC = 2
NS = 16
L = 16
NW = NC * NS
EPT = E // NW
G = 80
NCHUNK = EPT // G
AW = D + L
PW = D // 2
GW = PW + L
NZC = (N // G + NS - 1) // NS

NB = 2000
GRID = N // NB


def _tc_tail(i, g, avs_ref, avd_ref, gaug_ref, adst_ref, shift_ref, acc_ref):
    a_s = jnp.sum(g * avs_ref[...], axis=1, keepdims=True)
    a_d = jnp.sum(g * avd_ref[...], axis=1, keepdims=True)
    col = lax.broadcasted_iota(jnp.int32, (NB, L), 1)
    gb = g.astype(jnp.bfloat16)
    lo16 = lax.bitcast_convert_type(gb[:, :PW], jnp.uint16)
    hi16 = lax.bitcast_convert_type(gb[:, PW:], jnp.uint16)
    word = (hi16.astype(jnp.uint32) << 16) | lo16.astype(jnp.uint32)
    gaug_ref[:, :PW] = lax.bitcast_convert_type(word, jnp.float32)
    gaug_ref[:, PW:] = jnp.where(col == 0, 1.0,
                                 jnp.where(col == 1, a_s, 0.0))
    adst_ref[...] = jnp.broadcast_to(a_d, (NB, L))

    ma = jnp.max(a_s)
    md = jnp.max(a_d)

    @pl.when(i == 0)
    def _():
        acc_ref[0] = ma
        acc_ref[1] = md

    @pl.when(i > 0)
    def _():
        acc_ref[0] = jnp.maximum(acc_ref[0], ma)
        acc_ref[1] = jnp.maximum(acc_ref[1], md)

    @pl.when(i == GRID - 1)
    def _():
        s = acc_ref[0] + acc_ref[1]
        shift_ref[...] = jnp.maximum(s, 0.2 * s).reshape(1, 1)


_TC_OUT_SPECS = [
    pl.BlockSpec((NB, GW), lambda i: (i, 0)),
    pl.BlockSpec((NB, L), lambda i: (i, 0)),
    pl.BlockSpec((1, 1), lambda i: (0, 0)),
]
_TC_OUT_SHAPE = [
    jax.ShapeDtypeStruct((N, GW), jnp.float32),
    jax.ShapeDtypeStruct((N, L), jnp.float32),
    jax.ShapeDtypeStruct((1, 1), jnp.float32),
]


def _tc1_body(x_ref, y_ref, w1_ref, b1_ref, w2_ref, b2_ref, wg_ref,
              avs_ref, avd_ref,
              gaug_ref, adst_ref, shift_ref, acc_ref):
    i = pl.program_id(0)
    x = x_ref[...]
    h1 = jnp.maximum(x @ w1_ref[...] + b1_ref[...], 0.0)
    h2 = jnp.maximum(x @ w2_ref[...] + b2_ref[...], 0.0)
    h = jnp.where(y_ref[...] > 0.5, h1, h2)
    g = h @ wg_ref[...]
    _tc_tail(i, g, avs_ref, avd_ref, gaug_ref, adst_ref, shift_ref, acc_ref)


def _tc1(x, y_f, w1, b1, w2, b2, wg, avs, avd):
    row = lambda i: (i, 0)
    fixed = lambda i: (0, 0)
    return pl.pallas_call(
        _tc1_body,
        grid=(GRID,),
        in_specs=[
            pl.BlockSpec((NB, D), row),
            pl.BlockSpec((NB, 1), row),
            pl.BlockSpec((D, D), fixed),
            pl.BlockSpec((1, D), fixed),
            pl.BlockSpec((D, D), fixed),
            pl.BlockSpec((1, D), fixed),
            pl.BlockSpec((D, D), fixed),
            pl.BlockSpec((1, D), fixed),
            pl.BlockSpec((1, D), fixed),
        ],
        out_specs=_TC_OUT_SPECS,
        out_shape=_TC_OUT_SHAPE,
        scratch_shapes=[pltpu.SMEM((2,), jnp.float32)],
    )(x, y_f, w1, b1, w2, b2, wg, avs, avd)


def _tc2_body(accr_ref, bg_ref, wg_ref, avs_ref, avd_ref,
              gaug_ref, adst_ref, shift_ref, acc_ref):
    i = pl.program_id(0)
    p = accr_ref[0] + accr_ref[1]
    num = p[:, :D]
    den = p[:, D:D + 1]
    o = num / (den + 1e-16) + bg_ref[...]
    h = jnp.where(o > 0.0, o, jnp.exp(jnp.minimum(o, 0.0)) - 1.0)
    g = h @ wg_ref[...]
    _tc_tail(i, g, avs_ref, avd_ref, gaug_ref, adst_ref, shift_ref, acc_ref)


def _tc2(accr, bg, wg, avs, avd):
    fixed = lambda i: (0, 0)
    return pl.pallas_call(
        _tc2_body,
        grid=(GRID,),
        in_specs=[
            pl.BlockSpec((2, NB, AW), lambda i: (0, i, 0)),
            pl.BlockSpec((1, D), fixed),
            pl.BlockSpec((D, D), fixed),
            pl.BlockSpec((1, D), fixed),
            pl.BlockSpec((1, D), fixed),
        ],
        out_specs=_TC_OUT_SPECS,
        out_shape=_TC_OUT_SHAPE,
        scratch_shapes=[pltpu.SMEM((2,), jnp.float32)],
    )(accr, bg, wg, avs, avd)


def _tc3_body(accr_ref, bg_ref, out_ref):
    p = accr_ref[0] + accr_ref[1]
    num = p[:, :D]
    den = p[:, D:D + 1]
    o = num / (den + 1e-16) + bg_ref[...]
    out_ref[...] = jnp.where(o > 0.0, o, jnp.exp(jnp.minimum(o, 0.0)) - 1.0)


def _tc3(accr, bg):
    row = lambda i: (i, 0)
    fixed = lambda i: (0, 0)
    return pl.pallas_call(
        _tc3_body,
        grid=(GRID,),
        in_specs=[
            pl.BlockSpec((2, NB, AW), lambda i: (0, i, 0)),
            pl.BlockSpec((1, D), fixed),
        ],
        out_specs=pl.BlockSpec((NB, D), row),
        out_shape=jax.ShapeDtypeStruct((N, D), jnp.float32),
    )(accr, bg)


def _sc_body(gaug_h, adst_h, sd_h, shift_h,
             outr_h,
             grows0, grows1, stage0, stage1, avd0, avd1,
             sx0, sx1, sx2, sx3, sx4, sx5,
             dx0, dx1, dx2, dx3, dx4, dx5,
             ee_v, shift_v, accr_sh,
             sg0, sg1, sa0, sa1,
             si0, si1, si2, si3, si4, si5,
             ss0, ss1):
    grows = (grows0, grows1)
    stage = (stage0, stage1)
    avd = (avd0, avd1)
    sx = (sx0, sx1, sx2, sx3, sx4, sx5)
    dx = (dx0, dx1, dx2, dx3, dx4, dx5)
    sg = (sg0, sg1)
    sa = (sa0, sa1)
    si = (si0, si1, si2, si3, si4, si5)
    ss = (ss0, ss1)

    cid = lax.axis_index("c")
    sid = lax.axis_index("s")
    wid = sid * NC + cid

    pltpu.sync_copy(shift_h, shift_v)
    shift = shift_v[...]
    lane = lax.iota(jnp.int32, L)
    mask_hi = jnp.full((L,), -65536, jnp.int32)

    def _zrow(i, c):
        for k in range(AW // L):
            stage0[i, pl.ds(k * L, L)] = jnp.zeros((L,), jnp.float32)
        return c

    lax.fori_loop(0, G, _zrow, 0)
    for k in range(NZC):
        zi = sid + k * NS

        @pl.when(zi < N // G)
        def _():
            pltpu.sync_copy(stage0, accr_sh.at[pl.ds(zi * G, G)])

    plsc.subcore_barrier()

    def _issue_idx(j, k6):
        pltpu.async_copy(sd_h.at[0, wid, j], sx[k6], si[k6])
        pltpu.async_copy(sd_h.at[1, wid, j], dx[k6], si[k6])

    def _wait_idx(k6):
        pltpu.make_async_copy(sd_h.at[0, wid, 0], sx[k6], si[k6]).wait()
        pltpu.make_async_copy(sd_h.at[1, wid, 0], dx[k6], si[k6]).wait()

    def _issue_gather(k6, p2):
        pltpu.async_copy(gaug_h.at[sx[k6]], grows[p2], sg[p2])
        pltpu.async_copy(adst_h.at[dx[k6]], avd[p2], sa[p2])

    def _wait_gather(k6, p2):
        pltpu.make_async_copy(gaug_h.at[sx[k6]], grows[p2],
                              sg[p2]).wait()
        pltpu.make_async_copy(adst_h.at[dx[k6]], avd[p2],
                              sa[p2]).wait()

    def _wait_scatter(k6, p2):
        pltpu.make_async_copy(stage[p2], accr_sh.at[dx[k6]],
                              ss[p2]).wait()

    def _compute(p2):
        gr = grows[p2]
        av = avd[p2]
        st = stage[p2]
        for t in range(G // L):
            rowi = lane + t * L
            a1 = plsc.load_gather(gr, [rowi, lane * 0 + (PW + 1)])
            a2 = plsc.load_gather(av, [rowi, lane * 0])
            z = a1 + a2
            e = jnp.maximum(z, 0.2 * z)
            ee_v[pl.ds(t * L, L)] = jnp.exp(e - shift)

        @plsc.parallel_loop(0, G, step=1, unroll=4)
        def _row(r):
            sv = plsc.load_gather(ee_v, [jnp.full((L,), r, jnp.int32)])
            for k in range(PW // L):
                w = plsc.bitcast(gr[r, pl.ds(k * L, L)], jnp.int32)
                lo = plsc.bitcast(w << 16, jnp.float32)
                hi = plsc.bitcast(w & mask_hi, jnp.float32)
                st[r, pl.ds(k * L, L)] = lo * sv
                st[r, pl.ds(PW + k * L, L)] = hi * sv
            st[r, pl.ds(D, L)] = gr[r, pl.ds(PW, L)] * sv

    def _body(j, k6, p2):
        _wait_gather(k6, p2)

        @pl.when(j >= 2)
        def _():
            _wait_scatter((k6 + 4) % 6, p2)

        _compute(p2)
        pltpu.async_copy(stage[p2], accr_sh.at[dx[k6]], ss[p2],
                         add=True)
        _issue_idx(j + 4, (k6 + 4) % 6)
        _wait_idx((k6 + 2) % 6)
        _issue_gather((k6 + 2) % 6, p2)

    for p in range(2):
        pltpu.sync_copy(sd_h.at[0, wid, p], sx[p])
        pltpu.sync_copy(sd_h.at[1, wid, p], dx[p])
    for p in range(2, 4):
        _issue_idx(p, p)
    for p in range(2):
        _issue_gather(p, p)

    def _six(jo2, carry):
        for u in range(6):
            _body(jo2 * 6 + u, u, u % 2)
        return carry

    lax.fori_loop(0, 20, _six, 0)

    for j in range(120, NCHUNK):
        k6 = j % 6
        p2 = j % 2
        _wait_gather(k6, p2)
        _wait_scatter((k6 + 4) % 6, p2)
        _compute(p2)
        pltpu.async_copy(stage[p2], accr_sh.at[dx[k6]], ss[p2],
                         add=True)
        if j == 120:
            _issue_idx(j + 4, (k6 + 4) % 6)
        if j + 2 < NCHUNK:
            _wait_idx((k6 + 2) % 6)
            _issue_gather((k6 + 2) % 6, p2)

    _wait_scatter((NCHUNK - 2) % 6, (NCHUNK - 2) % 2)
    _wait_scatter((NCHUNK - 1) % 6, (NCHUNK - 1) % 2)
    plsc.subcore_barrier()

    for k in range(NZC):
        zi = sid + k * NS

        @pl.when(zi < N // G)
        def _():
            pltpu.sync_copy(accr_sh.at[pl.ds(zi * G, G)], stage0)
            pltpu.sync_copy(stage0, outr_h.at[cid, pl.ds(zi * G, G)])


def _sc_layer(gaug, adst16, sd, shift16):
    mesh = plsc.VectorSubcoreMesh(
        core_axis_name="c", subcore_axis_name="s", num_cores=NC,
        num_subcores=NS)
    f = pl.kernel(
        _sc_body,
        out_type=jax.ShapeDtypeStruct((NC, N, AW), jnp.float32),
        mesh=mesh,
        scratch_types=(
            [pltpu.VMEM((G, GW), jnp.float32)] * 2
            + [pltpu.VMEM((G, AW), jnp.float32)] * 2
            + [pltpu.VMEM((G, L), jnp.float32)] * 2
            + [pltpu.VMEM((G,), jnp.int32)] * 12
            + [pltpu.VMEM((G,), jnp.float32),
               pltpu.VMEM((L,), jnp.float32),
               pltpu.VMEM_SHARED((N, AW), jnp.float32)]
            + [pltpu.SemaphoreType.DMA] * 12
        ),
        compiler_params=pltpu.CompilerParams(
            use_tc_tiling_on_sc=False, needs_layout_passes=False),
    )
    return f(gaug, adst16, sd, shift16)


def kernel(x, edge_index, y, W1, b1, W2, b2,
           Wg1, as1, ad1, bg1, Wg2, as2, ad2, bg2):
    y_f = y.astype(jnp.float32).reshape(N, 1)
    sd = edge_index.astype(jnp.int32).reshape(2, NW, NCHUNK, G)

    gaug1, adst1, shift1 = _tc1(
        x, y_f, W1, b1.reshape(1, D), W2, b2.reshape(1, D), Wg1,
        as1.reshape(1, D), ad1.reshape(1, D))
    s16 = jnp.full((L,), shift1[0, 0], jnp.float32)
    accr1 = _sc_layer(gaug1, adst1, sd, s16)

    gaug2, adst2, shift2 = _tc2(
        accr1, bg1.reshape(1, D), Wg2, as2.reshape(1, D), ad2.reshape(1, D))
    s16b = jnp.full((L,), shift2[0, 0], jnp.float32)
    accr2 = _sc_layer(gaug2, adst2, sd, s16b)

    return _tc3(accr2, bg2.reshape(1, D))
